# Initial kernel scaffold; baseline (speedup 1.0000x reference)
import jax
import jax.numpy as jnp
from jax.experimental import pallas as pl


def kernel(encoded, embeddings):
    raise NotImplementedError("write your pallas kernel here")


# speedup vs baseline: 3.4157x; 3.4157x over previous
import functools

import jax
import jax.numpy as jnp
from jax import lax
from jax.experimental import pallas as pl
from jax.experimental.pallas import tpu as pltpu
from jax.experimental.pallas import tpu_sc as plsc

_B = 128
_D = 512
_N = _B * _D
_K = 2048
_NC = 2
_NS = 16
_L = 16
_NW = _NC * _NS
_EPW = _N // _NW
_CPS = _K // _NS
_NVR = _CPS // _L


@functools.partial(
    pl.kernel,
    out_type=(
        jax.ShapeDtypeStruct((_N,), jnp.float32),
        jax.ShapeDtypeStruct((_NW, _L), jnp.float32),
    ),
    mesh=plsc.VectorSubcoreMesh(core_axis_name="c", subcore_axis_name="s",
                                num_cores=_NC, num_subcores=_NS),
    compiler_params=pltpu.CompilerParams(needs_layout_passes=False),
    scratch_types=[
        pltpu.VMEM((_K,), jnp.float32),
        pltpu.VMEM((_CPS,), jnp.int32),
        pltpu.VMEM_SHARED((_K,), jnp.int32),
        pltpu.VMEM((_K,), jnp.int32),
        pltpu.VMEM((_K,), jnp.float32),
        pltpu.VMEM((_EPW,), jnp.float32),
        pltpu.VMEM((_EPW,), jnp.float32),
        pltpu.VMEM((_L,), jnp.float32),
    ],
)
def _vq_snap(x_hbm, emb_hbm, out_hbm, sq_hbm,
             emb_v, ranks_v, shr_ranks, ranks_all, sorted_v, x_v, o_v, acc_v):
    cid = lax.axis_index("c")
    sid = lax.axis_index("s")
    wid = sid * _NC + cid

    pltpu.sync_copy(emb_hbm, emb_v)
    i0 = sid * _CPS
    lanes = lax.iota(jnp.int32, _L)
    civ = [emb_v[pl.ds(i0 + r * _L, _L)] for r in range(_NVR)]
    iiv = [i0 + r * _L + lanes for r in range(_NVR)]
    one = jnp.ones((_L,), jnp.int32)
    zero = jnp.zeros((_L,), jnp.int32)

    jv0 = sid * _NVR

    def cnt_below(jv, cnts):
        cjv = emb_v[pl.ds(jv * _L, _L)]
        for lane in range(_L):
            cj = jnp.full((_L,), cjv[lane])
            cnts = tuple(c + jnp.where(cj <= cv, one, zero)
                         for c, cv in zip(cnts, civ))
        return cnts

    def cnt_mid(jv, cnts):
        cjv = emb_v[pl.ds(jv * _L, _L)]
        for lane in range(_L):
            cj = jnp.full((_L,), cjv[lane])
            j = jv * _L + lane
            out = []
            for c, cv, iv in zip(cnts, civ, iiv):
                hit = jnp.where(j < iv, cj <= cv, cj < cv)
                out.append(c + jnp.where(hit, one, zero))
            cnts = tuple(out)
        return cnts

    def cnt_above(jv, cnts):
        cjv = emb_v[pl.ds(jv * _L, _L)]
        for lane in range(_L):
            cj = jnp.full((_L,), cjv[lane])
            cnts = tuple(c + jnp.where(cj < cv, one, zero)
                         for c, cv in zip(cnts, civ))
        return cnts

    cnts = tuple(zero for _ in range(_NVR))
    cnts = lax.fori_loop(0, jv0, cnt_below, cnts)
    cnts = lax.fori_loop(jv0, jv0 + _NVR, cnt_mid, cnts)
    cnts = lax.fori_loop(jv0 + _NVR, _K // _L, cnt_above, cnts)
    for r in range(_NVR):
        ranks_v[pl.ds(r * _L, _L)] = cnts[r]

    pltpu.sync_copy(ranks_v, shr_ranks.at[pl.ds(i0, _CPS)])
    plsc.subcore_barrier()
    pltpu.sync_copy(shr_ranks, ranks_all)

    def scat(jv, carry):
        v = emb_v[pl.ds(jv * _L, _L)]
        r = ranks_all[pl.ds(jv * _L, _L)]
        plsc.store_scatter(sorted_v, [r], v)
        return carry

    lax.fori_loop(0, _K // _L, scat, 0)

    base = wid * _EPW
    pltpu.sync_copy(x_hbm.at[pl.ds(base, _EPW)], x_v)

    def search(v, acc):
        xv = x_v[pl.ds(v * _L, _L)]
        pos = jnp.zeros((_L,), jnp.int32)
        step = _K // 2
        while step >= 1:
            c = plsc.load_gather(sorted_v, [pos + (step - 1)])
            pos = jnp.where(c < xv, pos + step, pos)
            step //= 2
        i1 = jnp.maximum(pos - 1, 0)
        i2 = jnp.minimum(pos, _K - 1)
        a = plsc.load_gather(sorted_v, [i1])
        b = plsc.load_gather(sorted_v, [i2])
        da = (a - xv) * (a - xv)
        db = (b - xv) * (b - xv)
        lc = jnp.where(db < da, b, a)
        o_v[pl.ds(v * _L, _L)] = xv + (lc - xv)
        d = lc - xv
        return acc + d * d

    acc = lax.fori_loop(0, _EPW // _L, search,
                        jnp.zeros((_L,), jnp.float32))
    acc_v[...] = acc
    pltpu.sync_copy(o_v, out_hbm.at[pl.ds(base, _EPW)])
    pltpu.sync_copy(acc_v, sq_hbm.at[wid])


def kernel(encoded, embeddings):
    x = encoded.reshape(-1)
    emb = embeddings.reshape(-1)
    out, sq = _vq_snap(x, emb)
    latent_code_st = out.reshape(encoded.shape)
    loss = 2.0 * (jnp.sum(sq) / encoded.shape[0])
    return latent_code_st, loss
